# baseline (speedup 1.0000x reference)
import functools

import jax
import jax.numpy as jnp
from jax import lax
from jax.experimental import pallas as pl
from jax.experimental.pallas import tpu as pltpu
from jax.experimental.pallas import tpu_sc as plsc

_RADIUS2 = 0.8 * 0.8
_NS = 16
_B = 4
_NPB = 16384
_MPB = 4096
_N = _B * _NPB
_M = _B * _MPB
_C = 32
_L = 16
_QPW = _M // 32
_UNR = 16
_NSUP = _NPB // (_L * _UNR)
_CHUNK = 128
_ROWS = 3 + _C

_mesh = plsc.VectorSubcoreMesh(core_axis_name="c", subcore_axis_name="s")
_params = pltpu.CompilerParams(needs_layout_passes=False)


def _rtne_bits(v):
    u = plsc.bitcast(v, jnp.int32)
    return (u + jnp.int32(0x7FFF) + ((u >> 16) & 1)) & jnp.int32(-65536)


def _rtne_bf16(v):
    return plsc.bitcast(_rtne_bits(v), jnp.float32)


def _bcast(v, t):
    dn = lax.GatherDimensionNumbers(
        offset_dims=(), collapsed_slice_dims=(0,), start_index_map=(0,))
    tv = jnp.full((_L,), t, jnp.int32)
    return lax.gather(v, tv[:, None], dn, slice_sizes=(1,),
                      mode=lax.GatherScatterMode.PROMISE_IN_BOUNDS)


@functools.partial(
    pl.kernel,
    out_type=(
        jax.ShapeDtypeStruct((_M, _NS), jnp.int32),
        jax.ShapeDtypeStruct((_M, _NS), jnp.int32),
        jax.ShapeDtypeStruct((_M,), jnp.int32),
    ),
    mesh=_mesh,
    compiler_params=_params,
    scratch_types=[
        pltpu.VMEM((_NPB,), jnp.float32),
        pltpu.VMEM((_NPB,), jnp.float32),
        pltpu.VMEM((_NPB,), jnp.float32),
        pltpu.VMEM((_QPW,), jnp.float32),
        pltpu.VMEM((_QPW,), jnp.float32),
        pltpu.VMEM((_QPW,), jnp.float32),
        pltpu.VMEM((_NS,), jnp.int32),
        pltpu.VMEM((_L,), jnp.int32),
        pltpu.VMEM((64, _NS), jnp.int32),
        pltpu.VMEM((64, _NS), jnp.int32),
        pltpu.VMEM((_QPW,), jnp.int32),
        pltpu.SMEM((1,), jnp.int32),
    ],
)
def _ball_query(xyzt, nxyzt, idx_out, idx2_out, cnt_out,
                pp, xy, zs, qx, qy, qz, acc, cntref, idxb, idxb2, cntb, cs_s):
    wid = lax.axis_index("c") * 16 + lax.axis_index("s")
    qbase = wid * _QPW
    seg = qbase // _MPB
    pbase = seg * _NPB
    pltpu.sync_copy(xyzt.at[pl.ds(0 * _N + pbase, _NPB)], pp)
    pltpu.sync_copy(xyzt.at[pl.ds(1 * _N + pbase, _NPB)], xy)
    pltpu.sync_copy(xyzt.at[pl.ds(2 * _N + pbase, _NPB)], zs)
    pltpu.sync_copy(nxyzt.at[pl.ds(0 * _M + qbase, _QPW)], qx)
    pltpu.sync_copy(nxyzt.at[pl.ds(1 * _M + qbase, _QPW)], qy)
    pltpu.sync_copy(nxyzt.at[pl.ds(2 * _M + qbase, _QPW)], qz)

    iota = lax.iota(jnp.int32, _L)
    lane0 = iota == 0
    zeros = jnp.zeros((_L,), jnp.int32)

    def prep(i, carry_unused):
        b = pl.multiple_of(i * _L, _L)
        vx = pp[pl.ds(b, _L)]
        vy = xy[pl.ds(b, _L)]
        vz = zs[pl.ds(b, _L)]
        pp[pl.ds(b, _L)] = vx * vx + vy * vy + vz * vz
        bx = _rtne_bits(vx)
        by = _rtne_bits(vy)
        xy[pl.ds(b, _L)] = plsc.bitcast(
            bx | ((by >> 16) & jnp.int32(0xFFFF)), jnp.float32)
        zs[pl.ds(b, _L)] = _rtne_bf16(vz)
        return 0

    lax.fori_loop(0, _NPB // _L, prep, 0)

    def per_query(g, s, t):
        q = g * 64 + s * _L + t
        base16 = g * 64 + s * _L
        qx16 = qx[pl.ds(base16, _L)]
        qy16 = qy[pl.ds(base16, _L)]
        qz16 = qz[pl.ds(base16, _L)]
        qq16 = qx16 * qx16 + qy16 * qy16 + qz16 * qz16
        qq = _bcast(qq16, t)
        qxb = _bcast(_rtne_bf16(qx16), t)
        qyb = _bcast(_rtne_bf16(qy16), t)
        qzb = _bcast(_rtne_bf16(qz16), t)
        cntref[...] = zeros
        cs_s[0] = jnp.int32(0)

        def within_at(base):
            w = plsc.bitcast(xy[pl.ds(base, _L)], jnp.int32)
            px = plsc.bitcast(w & jnp.int32(-65536), jnp.float32)
            py = plsc.bitcast(w << 16, jnp.float32)
            pz = zs[pl.ds(base, _L)]
            dot = qxb * px + qyb * py + qzb * pz
            d2 = (qq + pp[pl.ds(base, _L)]) - jnp.float32(2.0) * dot
            return d2 < _RADIUS2

        def cond(carry):
            j, cs = carry
            return jnp.logical_and(j < _NSUP, cs < _NS)

        def body(carry):
            j, cs = carry
            sup = pl.multiple_of(j * (_L * _UNR), _L * _UNR)
            anym = within_at(sup)
            for u in range(1, _UNR):
                anym = jnp.logical_or(anym, within_at(sup + u * _L))

            @pl.when(jnp.any(anym))
            def _append():
                cntv = cntref[...]
                for u in range(_UNR):
                    base = sup + u * _L
                    within = within_at(base)
                    c = plsc.all_reduce_population_count(within)
                    keys = base + iota
                    sk = plsc.sort_key_val(keys, keys, mask=within)[0]
                    dest = cntv + iota
                    m = jnp.logical_and(iota < c, dest < _NS)
                    plsc.store_scatter(acc, [dest], sk, mask=m)
                    cntv = jnp.minimum(cntv + c, _NS)
                cntref[...] = cntv
                cs_s[0] = jnp.max(cntv)

            return j + 1, cs_s[0]

        lax.while_loop(cond, body, (jnp.int32(0), jnp.int32(0)))
        cntv = cntref[...]
        accv = acc[...]
        first = plsc.cummax(jnp.where(lane0, accv, jnp.int32(-1)))
        valid = iota < cntv
        nonempty = cntv > 0
        idxv = jnp.where(nonempty, jnp.where(valid, accv, first), 0)
        tl = s * _L + t
        idxb[tl, :] = idxv
        idxb2[tl, :] = jnp.where(nonempty, idxv, _NPB)
        qsplat = jnp.full((_L,), q, jnp.int32)
        plsc.store_scatter(cntb, [qsplat], cntv, mask=lane0)
        return 0

    def per_group(g, carry_unused):
        for s in range(4):
            lax.fori_loop(0, _L, lambda t, cu: per_query(g, s, t), 0)
        pltpu.sync_copy(idxb, idx_out.at[pl.ds(qbase + g * 64, 64)])
        pltpu.sync_copy(idxb2, idx2_out.at[pl.ds(qbase + g * 64, 64)])
        return 0

    lax.fori_loop(0, _QPW // 64, per_group, 0)
    pltpu.sync_copy(cntb, cnt_out.at[pl.ds(qbase, _QPW)])


@functools.partial(
    pl.kernel,
    out_type=jax.ShapeDtypeStruct((_ROWS, _M, _NS), jnp.float32),
    mesh=_mesh,
    compiler_params=_params,
    scratch_types=[
        pltpu.VMEM((_NPB + _L,), jnp.float32),
        pltpu.VMEM((_NPB + _L,), jnp.float32),
        pltpu.VMEM((_MPB,), jnp.float32),
        pltpu.VMEM((_CHUNK, _NS), jnp.int32),
        pltpu.VMEM((_CHUNK,), jnp.int32),
        pltpu.VMEM((_CHUNK, _L), jnp.float32),
        pltpu.VMEM((_CHUNK, _L), jnp.float32),
        pltpu.SemaphoreType.DMA,
    ],
)
def _group(table, cent, idx_in, cnt_in, out,
           ta, tb, ccol, idxc, cntc, sa, sb, sem):
    wid = lax.axis_index("c") * 16 + lax.axis_index("s")
    seg = wid // 8
    k = wid % 8
    qg0 = seg * _MPB
    pbase = seg * _NPB
    fzeros = jnp.zeros((_L,), jnp.float32)

    def load_col(chan, buf):
        pltpu.sync_copy(table.at[pl.ds(chan * _N + pbase, _NPB)],
                        buf.at[pl.ds(0, _NPB)])
        buf[pl.ds(_NPB, _L)] = fzeros

    def run_pass2(chA, chB, coord):
        load_col(chA, ta)
        load_col(chB, tb)
        if coord:
            pltpu.sync_copy(cent.at[pl.ds(chA * _M + qg0, _MPB)], ccol)

        def per_chunk(cb, carry_unused):
            qb = qg0 + cb * _CHUNK
            pltpu.sync_copy(idx_in.at[pl.ds(qb, _CHUNK)], idxc)
            if coord:
                pltpu.sync_copy(cnt_in.at[pl.ds(qb, _CHUNK)], cntc)

            def do_query(q):
                idxv = idxc[q, :]
                va = plsc.load_gather(ta, [idxv])
                if coord:
                    qsplat = jnp.full((_L,), q, jnp.int32)
                    cq = plsc.load_gather(cntc, [qsplat])
                    qcs = jnp.full((_L,), cb * _CHUNK, jnp.int32) + qsplat
                    cv = plsc.load_gather(ccol, [qcs])
                    va = jnp.where(cq > 0, va - cv, jnp.float32(0.0))
                sa[q, :] = va
                sb[q, :] = plsc.load_gather(tb, [idxv])

            def per_q4(qu, carry_unused2):
                q0 = qu * 4
                for i in range(4):
                    do_query(q0 + i)
                return 0

            lax.fori_loop(0, _CHUNK // 4, per_q4, 0)
            d1 = pltpu.async_copy(sa, out.at[chA, pl.ds(qb, _CHUNK), :], sem)
            d2 = pltpu.async_copy(sb, out.at[chB, pl.ds(qb, _CHUNK), :], sem)
            d1.wait()
            d2.wait()
            return 0

        lax.fori_loop(0, _MPB // _CHUNK, per_chunk, 0)

    def run_pass1(chA):
        load_col(chA, ta)

        def per_chunk(cb, carry_unused):
            qb = qg0 + cb * _CHUNK
            pltpu.sync_copy(idx_in.at[pl.ds(qb, _CHUNK)], idxc)

            def per_q4(qu, carry_unused2):
                q0 = qu * 4
                for i in range(4):
                    sa[q0 + i, :] = plsc.load_gather(ta, [idxc[q0 + i, :]])
                return 0

            lax.fori_loop(0, _CHUNK // 4, per_q4, 0)
            pltpu.sync_copy(sa, out.at[chA, pl.ds(qb, _CHUNK), :])
            return 0

        lax.fori_loop(0, _MPB // _CHUNK, per_chunk, 0)

    @pl.when(k < 3)
    def _coord():
        run_pass2(k, k + 8, coord=True)

    @pl.when(k >= 3)
    def _plain():
        run_pass2(k, k + 8, coord=False)

    run_pass2(k + 16, k + 24, coord=False)

    @pl.when(k < 3)
    def _tail():
        run_pass1(k + 32)


def kernel(xyz, xyz_batch_cnt, new_xyz, new_xyz_batch_cnt, features):
    xyzt = jnp.transpose(xyz).reshape(-1)
    nxyzt = jnp.transpose(new_xyz).reshape(-1)
    idx, idx2, cnt = _ball_query(xyzt, nxyzt)
    table = jnp.concatenate(
        [jnp.transpose(xyz), jnp.transpose(features)], axis=0).reshape(-1)
    cent = nxyzt
    grouped = _group(table, cent, idx2, cnt)
    new_features = jnp.transpose(grouped, (1, 0, 2))
    return new_features, idx


# speedup vs baseline: 1.0390x; 1.0390x over previous
import functools

import jax
import jax.numpy as jnp
from jax import lax
from jax.experimental import pallas as pl
from jax.experimental.pallas import tpu as pltpu
from jax.experimental.pallas import tpu_sc as plsc

_RADIUS2 = 0.8 * 0.8
_NS = 16
_B = 4
_NPB = 16384
_MPB = 4096
_N = _B * _NPB
_M = _B * _MPB
_C = 32
_L = 16
_QPW = _M // 32
_UNR = 4
_NSUP = _NPB // (_L * _UNR)
_CHUNK = 128
_ROWS = 3 + _C

_mesh = plsc.VectorSubcoreMesh(core_axis_name="c", subcore_axis_name="s")
_params = pltpu.CompilerParams(needs_layout_passes=False)


def _rtne_bits(v):
    u = plsc.bitcast(v, jnp.int32)
    return (u + jnp.int32(0x7FFF) + ((u >> 16) & 1)) & jnp.int32(-65536)


def _rtne_bf16(v):
    return plsc.bitcast(_rtne_bits(v), jnp.float32)


def _bcast(v, t):
    dn = lax.GatherDimensionNumbers(
        offset_dims=(), collapsed_slice_dims=(0,), start_index_map=(0,))
    tv = jnp.full((_L,), t, jnp.int32)
    return lax.gather(v, tv[:, None], dn, slice_sizes=(1,),
                      mode=lax.GatherScatterMode.PROMISE_IN_BOUNDS)


@functools.partial(
    pl.kernel,
    out_type=(
        jax.ShapeDtypeStruct((_M, _NS), jnp.int32),
        jax.ShapeDtypeStruct((_M, _NS), jnp.int32),
        jax.ShapeDtypeStruct((_M,), jnp.int32),
    ),
    mesh=_mesh,
    compiler_params=_params,
    scratch_types=[
        pltpu.VMEM((_NPB,), jnp.float32),
        pltpu.VMEM((_NPB,), jnp.float32),
        pltpu.VMEM((_NPB,), jnp.float32),
        pltpu.VMEM((_QPW,), jnp.float32),
        pltpu.VMEM((_QPW,), jnp.float32),
        pltpu.VMEM((_QPW,), jnp.float32),
        pltpu.VMEM((_NS,), jnp.int32),
        pltpu.VMEM((_L,), jnp.int32),
        pltpu.VMEM((64, _NS), jnp.int32),
        pltpu.VMEM((64, _NS), jnp.int32),
        pltpu.VMEM((_QPW,), jnp.int32),
        pltpu.SMEM((1,), jnp.int32),
    ],
)
def _ball_query(xyzt, nxyzt, idx_out, idx2_out, cnt_out,
                pp, xy, zs, qx, qy, qz, acc, cntref, idxb, idxb2, cntb, cs_s):
    wid = lax.axis_index("c") * 16 + lax.axis_index("s")
    qbase = wid * _QPW
    seg = qbase // _MPB
    pbase = seg * _NPB
    pltpu.sync_copy(xyzt.at[pl.ds(0 * _N + pbase, _NPB)], pp)
    pltpu.sync_copy(xyzt.at[pl.ds(1 * _N + pbase, _NPB)], xy)
    pltpu.sync_copy(xyzt.at[pl.ds(2 * _N + pbase, _NPB)], zs)
    pltpu.sync_copy(nxyzt.at[pl.ds(0 * _M + qbase, _QPW)], qx)
    pltpu.sync_copy(nxyzt.at[pl.ds(1 * _M + qbase, _QPW)], qy)
    pltpu.sync_copy(nxyzt.at[pl.ds(2 * _M + qbase, _QPW)], qz)

    iota = lax.iota(jnp.int32, _L)
    lane0 = iota == 0
    zeros = jnp.zeros((_L,), jnp.int32)

    def prep(i, carry_unused):
        b = pl.multiple_of(i * _L, _L)
        vx = pp[pl.ds(b, _L)]
        vy = xy[pl.ds(b, _L)]
        vz = zs[pl.ds(b, _L)]
        pp[pl.ds(b, _L)] = vx * vx + vy * vy + vz * vz
        bx = _rtne_bits(vx)
        by = _rtne_bits(vy)
        xy[pl.ds(b, _L)] = plsc.bitcast(
            bx | ((by >> 16) & jnp.int32(0xFFFF)), jnp.float32)
        zs[pl.ds(b, _L)] = _rtne_bf16(vz)
        return 0

    lax.fori_loop(0, _NPB // _L, prep, 0)

    def per_query(g, s, t):
        q = g * 64 + s * _L + t
        base16 = g * 64 + s * _L
        qx16 = qx[pl.ds(base16, _L)]
        qy16 = qy[pl.ds(base16, _L)]
        qz16 = qz[pl.ds(base16, _L)]
        qq16 = qx16 * qx16 + qy16 * qy16 + qz16 * qz16
        qq = _bcast(qq16, t)
        qxb = _bcast(_rtne_bf16(qx16), t)
        qyb = _bcast(_rtne_bf16(qy16), t)
        qzb = _bcast(_rtne_bf16(qz16), t)
        cntref[...] = zeros
        cs_s[0] = jnp.int32(0)

        def within_at(base):
            w = plsc.bitcast(xy[pl.ds(base, _L)], jnp.int32)
            px = plsc.bitcast(w & jnp.int32(-65536), jnp.float32)
            py = plsc.bitcast(w << 16, jnp.float32)
            pz = zs[pl.ds(base, _L)]
            dot = qxb * px + qyb * py + qzb * pz
            d2 = (qq + pp[pl.ds(base, _L)]) - jnp.float32(2.0) * dot
            return d2 < _RADIUS2

        def cond(carry):
            j, cs = carry
            return jnp.logical_and(j < _NSUP, cs < _NS)

        def body(carry):
            j, cs = carry
            sup = pl.multiple_of(j * (_L * _UNR), _L * _UNR)
            anym = within_at(sup)
            for u in range(1, _UNR):
                anym = jnp.logical_or(anym, within_at(sup + u * _L))

            @pl.when(jnp.any(anym))
            def _append():
                cntv = cntref[...]
                for u in range(_UNR):
                    base = sup + u * _L
                    within = within_at(base)
                    c = plsc.all_reduce_population_count(within)
                    keys = base + iota
                    sk = plsc.sort_key_val(keys, keys, mask=within)[0]
                    dest = cntv + iota
                    m = jnp.logical_and(iota < c, dest < _NS)
                    plsc.store_scatter(acc, [dest], sk, mask=m)
                    cntv = jnp.minimum(cntv + c, _NS)
                cntref[...] = cntv
                cs_s[0] = jnp.max(cntv)

            return j + 1, cs_s[0]

        lax.while_loop(cond, body, (jnp.int32(0), jnp.int32(0)))
        cntv = cntref[...]
        accv = acc[...]
        first = plsc.cummax(jnp.where(lane0, accv, jnp.int32(-1)))
        valid = iota < cntv
        nonempty = cntv > 0
        idxv = jnp.where(nonempty, jnp.where(valid, accv, first), 0)
        tl = s * _L + t
        idxb[tl, :] = idxv
        idxb2[tl, :] = jnp.where(nonempty, idxv, _NPB)
        qsplat = jnp.full((_L,), q, jnp.int32)
        plsc.store_scatter(cntb, [qsplat], cntv, mask=lane0)
        return 0

    def per_group(g, carry_unused):
        for s in range(4):
            lax.fori_loop(0, _L, lambda t, cu: per_query(g, s, t), 0)
        pltpu.sync_copy(idxb, idx_out.at[pl.ds(qbase + g * 64, 64)])
        pltpu.sync_copy(idxb2, idx2_out.at[pl.ds(qbase + g * 64, 64)])
        return 0

    lax.fori_loop(0, _QPW // 64, per_group, 0)
    pltpu.sync_copy(cntb, cnt_out.at[pl.ds(qbase, _QPW)])


@functools.partial(
    pl.kernel,
    out_type=jax.ShapeDtypeStruct((_ROWS, _M, _NS), jnp.float32),
    mesh=_mesh,
    compiler_params=_params,
    scratch_types=[
        pltpu.VMEM((_NPB + _L,), jnp.float32),
        pltpu.VMEM((_NPB + _L,), jnp.float32),
        pltpu.VMEM((_MPB,), jnp.float32),
        pltpu.VMEM((_CHUNK, _NS), jnp.int32),
        pltpu.VMEM((_CHUNK,), jnp.int32),
        pltpu.VMEM((_CHUNK, _L), jnp.float32),
        pltpu.VMEM((_CHUNK, _L), jnp.float32),
        pltpu.SemaphoreType.DMA,
    ],
)
def _group(table, cent, idx_in, cnt_in, out,
           ta, tb, ccol, idxc, cntc, sa, sb, sem):
    wid = lax.axis_index("c") * 16 + lax.axis_index("s")
    seg = wid // 8
    k = wid % 8
    qg0 = seg * _MPB
    pbase = seg * _NPB
    fzeros = jnp.zeros((_L,), jnp.float32)

    def load_col(chan, buf):
        pltpu.sync_copy(table.at[pl.ds(chan * _N + pbase, _NPB)],
                        buf.at[pl.ds(0, _NPB)])
        buf[pl.ds(_NPB, _L)] = fzeros

    def run_pass2(chA, chB, coord):
        load_col(chA, ta)
        load_col(chB, tb)
        if coord:
            pltpu.sync_copy(cent.at[pl.ds(chA * _M + qg0, _MPB)], ccol)

        def per_chunk(cb, carry_unused):
            qb = qg0 + cb * _CHUNK
            pltpu.sync_copy(idx_in.at[pl.ds(qb, _CHUNK)], idxc)
            if coord:
                pltpu.sync_copy(cnt_in.at[pl.ds(qb, _CHUNK)], cntc)

            def do_query(q):
                idxv = idxc[q, :]
                va = plsc.load_gather(ta, [idxv])
                if coord:
                    qsplat = jnp.full((_L,), q, jnp.int32)
                    cq = plsc.load_gather(cntc, [qsplat])
                    qcs = jnp.full((_L,), cb * _CHUNK, jnp.int32) + qsplat
                    cv = plsc.load_gather(ccol, [qcs])
                    va = jnp.where(cq > 0, va - cv, jnp.float32(0.0))
                sa[q, :] = va
                sb[q, :] = plsc.load_gather(tb, [idxv])

            def per_q4(qu, carry_unused2):
                q0 = qu * 4
                for i in range(4):
                    do_query(q0 + i)
                return 0

            lax.fori_loop(0, _CHUNK // 4, per_q4, 0)
            d1 = pltpu.async_copy(sa, out.at[chA, pl.ds(qb, _CHUNK), :], sem)
            d2 = pltpu.async_copy(sb, out.at[chB, pl.ds(qb, _CHUNK), :], sem)
            d1.wait()
            d2.wait()
            return 0

        lax.fori_loop(0, _MPB // _CHUNK, per_chunk, 0)

    def run_pass1(chA):
        load_col(chA, ta)

        def per_chunk(cb, carry_unused):
            qb = qg0 + cb * _CHUNK
            pltpu.sync_copy(idx_in.at[pl.ds(qb, _CHUNK)], idxc)

            def per_q4(qu, carry_unused2):
                q0 = qu * 4
                for i in range(4):
                    sa[q0 + i, :] = plsc.load_gather(ta, [idxc[q0 + i, :]])
                return 0

            lax.fori_loop(0, _CHUNK // 4, per_q4, 0)
            pltpu.sync_copy(sa, out.at[chA, pl.ds(qb, _CHUNK), :])
            return 0

        lax.fori_loop(0, _MPB // _CHUNK, per_chunk, 0)

    @pl.when(k < 3)
    def _coord():
        run_pass2(k, k + 8, coord=True)

    @pl.when(k >= 3)
    def _plain():
        run_pass2(k, k + 8, coord=False)

    run_pass2(k + 16, k + 24, coord=False)

    @pl.when(k < 3)
    def _tail():
        run_pass1(k + 32)


def kernel(xyz, xyz_batch_cnt, new_xyz, new_xyz_batch_cnt, features):
    xyzt = jnp.transpose(xyz).reshape(-1)
    nxyzt = jnp.transpose(new_xyz).reshape(-1)
    idx, idx2, cnt = _ball_query(xyzt, nxyzt)
    table = jnp.concatenate(
        [jnp.transpose(xyz), jnp.transpose(features)], axis=0).reshape(-1)
    cent = nxyzt
    grouped = _group(table, cent, idx2, cnt)
    new_features = jnp.transpose(grouped, (1, 0, 2))
    return new_features, idx


# speedup vs baseline: 1.0493x; 1.0098x over previous
import functools

import jax
import jax.numpy as jnp
from jax import lax
from jax.experimental import pallas as pl
from jax.experimental.pallas import tpu as pltpu
from jax.experimental.pallas import tpu_sc as plsc

_RADIUS2 = 0.8 * 0.8
_NS = 16
_B = 4
_NPB = 16384
_MPB = 4096
_N = _B * _NPB
_M = _B * _MPB
_C = 32
_L = 16
_QPW = _M // 32
_UNR = 8
_NSUP = _NPB // (_L * _UNR)
_CHUNK = 128
_ROWS = 3 + _C

_mesh = plsc.VectorSubcoreMesh(core_axis_name="c", subcore_axis_name="s")
_params = pltpu.CompilerParams(needs_layout_passes=False)


def _rtne_bits(v):
    u = plsc.bitcast(v, jnp.int32)
    return (u + jnp.int32(0x7FFF) + ((u >> 16) & 1)) & jnp.int32(-65536)


def _rtne_bf16(v):
    return plsc.bitcast(_rtne_bits(v), jnp.float32)


def _bcast(v, t):
    dn = lax.GatherDimensionNumbers(
        offset_dims=(), collapsed_slice_dims=(0,), start_index_map=(0,))
    tv = jnp.full((_L,), t, jnp.int32)
    return lax.gather(v, tv[:, None], dn, slice_sizes=(1,),
                      mode=lax.GatherScatterMode.PROMISE_IN_BOUNDS)


@functools.partial(
    pl.kernel,
    out_type=(
        jax.ShapeDtypeStruct((_M, _NS), jnp.int32),
        jax.ShapeDtypeStruct((_M, _NS), jnp.int32),
        jax.ShapeDtypeStruct((_M,), jnp.int32),
    ),
    mesh=_mesh,
    compiler_params=_params,
    scratch_types=[
        pltpu.VMEM((_NPB,), jnp.float32),
        pltpu.VMEM((_NPB,), jnp.float32),
        pltpu.VMEM((_NPB,), jnp.float32),
        pltpu.VMEM((_QPW,), jnp.float32),
        pltpu.VMEM((_QPW,), jnp.float32),
        pltpu.VMEM((_QPW,), jnp.float32),
        pltpu.VMEM((_NS,), jnp.int32),
        pltpu.VMEM((_L,), jnp.int32),
        pltpu.VMEM((64, _NS), jnp.int32),
        pltpu.VMEM((64, _NS), jnp.int32),
        pltpu.VMEM((_QPW,), jnp.int32),
        pltpu.SMEM((1,), jnp.int32),
    ],
)
def _ball_query(xyzt, nxyzt, idx_out, idx2_out, cnt_out,
                pp, xy, zs, qx, qy, qz, acc, cntref, idxb, idxb2, cntb, cs_s):
    wid = lax.axis_index("c") * 16 + lax.axis_index("s")
    qbase = wid * _QPW
    seg = qbase // _MPB
    pbase = seg * _NPB
    pltpu.sync_copy(xyzt.at[pl.ds(0 * _N + pbase, _NPB)], pp)
    pltpu.sync_copy(xyzt.at[pl.ds(1 * _N + pbase, _NPB)], xy)
    pltpu.sync_copy(xyzt.at[pl.ds(2 * _N + pbase, _NPB)], zs)
    pltpu.sync_copy(nxyzt.at[pl.ds(0 * _M + qbase, _QPW)], qx)
    pltpu.sync_copy(nxyzt.at[pl.ds(1 * _M + qbase, _QPW)], qy)
    pltpu.sync_copy(nxyzt.at[pl.ds(2 * _M + qbase, _QPW)], qz)

    iota = lax.iota(jnp.int32, _L)
    lane0 = iota == 0
    zeros = jnp.zeros((_L,), jnp.int32)

    def prep(i, carry_unused):
        b = pl.multiple_of(i * _L, _L)
        vx = pp[pl.ds(b, _L)]
        vy = xy[pl.ds(b, _L)]
        vz = zs[pl.ds(b, _L)]
        pp[pl.ds(b, _L)] = vx * vx + vy * vy + vz * vz
        bx = _rtne_bits(vx)
        by = _rtne_bits(vy)
        xy[pl.ds(b, _L)] = plsc.bitcast(
            bx | ((by >> 16) & jnp.int32(0xFFFF)), jnp.float32)
        zs[pl.ds(b, _L)] = _rtne_bf16(vz)
        return 0

    lax.fori_loop(0, _NPB // _L, prep, 0)

    def per_query(g, s, t):
        q = g * 64 + s * _L + t
        base16 = g * 64 + s * _L
        qx16 = qx[pl.ds(base16, _L)]
        qy16 = qy[pl.ds(base16, _L)]
        qz16 = qz[pl.ds(base16, _L)]
        qq16 = qx16 * qx16 + qy16 * qy16 + qz16 * qz16
        qq = _bcast(qq16, t)
        qxb = _bcast(_rtne_bf16(qx16), t)
        qyb = _bcast(_rtne_bf16(qy16), t)
        qzb = _bcast(_rtne_bf16(qz16), t)
        cntref[...] = zeros
        cs_s[0] = jnp.int32(0)

        def within_at(base):
            w = plsc.bitcast(xy[pl.ds(base, _L)], jnp.int32)
            px = plsc.bitcast(w & jnp.int32(-65536), jnp.float32)
            py = plsc.bitcast(w << 16, jnp.float32)
            pz = zs[pl.ds(base, _L)]
            dot = qxb * px + qyb * py + qzb * pz
            d2 = (qq + pp[pl.ds(base, _L)]) - jnp.float32(2.0) * dot
            return d2 < _RADIUS2

        def cond(carry):
            j, cs = carry
            return jnp.logical_and(j < _NSUP, cs < _NS)

        def body(carry):
            j, cs = carry
            sup = pl.multiple_of(j * (_L * _UNR), _L * _UNR)
            anym = within_at(sup)
            for u in range(1, _UNR):
                anym = jnp.logical_or(anym, within_at(sup + u * _L))

            @pl.when(jnp.any(anym))
            def _append():
                cntv = cntref[...]
                for u in range(_UNR):
                    base = sup + u * _L
                    within = within_at(base)
                    c = plsc.all_reduce_population_count(within)
                    keys = base + iota
                    sk = plsc.sort_key_val(keys, keys, mask=within)[0]
                    dest = cntv + iota
                    m = jnp.logical_and(iota < c, dest < _NS)
                    plsc.store_scatter(acc, [dest], sk, mask=m)
                    cntv = jnp.minimum(cntv + c, _NS)
                cntref[...] = cntv
                cs_s[0] = jnp.max(cntv)

            return j + 1, cs_s[0]

        lax.while_loop(cond, body, (jnp.int32(0), jnp.int32(0)))
        cntv = cntref[...]
        accv = acc[...]
        first = plsc.cummax(jnp.where(lane0, accv, jnp.int32(-1)))
        valid = iota < cntv
        nonempty = cntv > 0
        idxv = jnp.where(nonempty, jnp.where(valid, accv, first), 0)
        tl = s * _L + t
        idxb[tl, :] = idxv
        idxb2[tl, :] = jnp.where(nonempty, idxv, _NPB)
        qsplat = jnp.full((_L,), q, jnp.int32)
        plsc.store_scatter(cntb, [qsplat], cntv, mask=lane0)
        return 0

    def per_group(g, carry_unused):
        for s in range(4):
            lax.fori_loop(0, _L, lambda t, cu: per_query(g, s, t), 0)
        pltpu.sync_copy(idxb, idx_out.at[pl.ds(qbase + g * 64, 64)])
        pltpu.sync_copy(idxb2, idx2_out.at[pl.ds(qbase + g * 64, 64)])
        return 0

    lax.fori_loop(0, _QPW // 64, per_group, 0)
    pltpu.sync_copy(cntb, cnt_out.at[pl.ds(qbase, _QPW)])


@functools.partial(
    pl.kernel,
    out_type=jax.ShapeDtypeStruct((_ROWS, _M, _NS), jnp.float32),
    mesh=_mesh,
    compiler_params=_params,
    scratch_types=[
        pltpu.VMEM((_NPB + _L,), jnp.float32),
        pltpu.VMEM((_NPB + _L,), jnp.float32),
        pltpu.VMEM((_MPB,), jnp.float32),
        pltpu.VMEM((_CHUNK, _NS), jnp.int32),
        pltpu.VMEM((_CHUNK,), jnp.int32),
        pltpu.VMEM((_CHUNK, _L), jnp.float32),
        pltpu.VMEM((_CHUNK, _L), jnp.float32),
        pltpu.SemaphoreType.DMA,
    ],
)
def _group(table, cent, idx_in, cnt_in, out,
           ta, tb, ccol, idxc, cntc, sa, sb, sem):
    wid = lax.axis_index("c") * 16 + lax.axis_index("s")
    seg = wid // 8
    k = wid % 8
    qg0 = seg * _MPB
    pbase = seg * _NPB
    fzeros = jnp.zeros((_L,), jnp.float32)

    def load_col(chan, buf):
        pltpu.sync_copy(table.at[pl.ds(chan * _N + pbase, _NPB)],
                        buf.at[pl.ds(0, _NPB)])
        buf[pl.ds(_NPB, _L)] = fzeros

    def run_pass2(chA, chB, coord):
        load_col(chA, ta)
        load_col(chB, tb)
        if coord:
            pltpu.sync_copy(cent.at[pl.ds(chA * _M + qg0, _MPB)], ccol)

        def per_chunk(cb, carry_unused):
            qb = qg0 + cb * _CHUNK
            pltpu.sync_copy(idx_in.at[pl.ds(qb, _CHUNK)], idxc)
            if coord:
                pltpu.sync_copy(cnt_in.at[pl.ds(qb, _CHUNK)], cntc)

            def do_query(q):
                idxv = idxc[q, :]
                va = plsc.load_gather(ta, [idxv])
                if coord:
                    qsplat = jnp.full((_L,), q, jnp.int32)
                    cq = plsc.load_gather(cntc, [qsplat])
                    qcs = jnp.full((_L,), cb * _CHUNK, jnp.int32) + qsplat
                    cv = plsc.load_gather(ccol, [qcs])
                    va = jnp.where(cq > 0, va - cv, jnp.float32(0.0))
                sa[q, :] = va
                sb[q, :] = plsc.load_gather(tb, [idxv])

            def per_q4(qu, carry_unused2):
                q0 = qu * 4
                for i in range(4):
                    do_query(q0 + i)
                return 0

            lax.fori_loop(0, _CHUNK // 4, per_q4, 0)
            d1 = pltpu.async_copy(sa, out.at[chA, pl.ds(qb, _CHUNK), :], sem)
            d2 = pltpu.async_copy(sb, out.at[chB, pl.ds(qb, _CHUNK), :], sem)
            d1.wait()
            d2.wait()
            return 0

        lax.fori_loop(0, _MPB // _CHUNK, per_chunk, 0)

    def run_pass1(chA):
        load_col(chA, ta)

        def per_chunk(cb, carry_unused):
            qb = qg0 + cb * _CHUNK
            pltpu.sync_copy(idx_in.at[pl.ds(qb, _CHUNK)], idxc)

            def per_q4(qu, carry_unused2):
                q0 = qu * 4
                for i in range(4):
                    sa[q0 + i, :] = plsc.load_gather(ta, [idxc[q0 + i, :]])
                return 0

            lax.fori_loop(0, _CHUNK // 4, per_q4, 0)
            pltpu.sync_copy(sa, out.at[chA, pl.ds(qb, _CHUNK), :])
            return 0

        lax.fori_loop(0, _MPB // _CHUNK, per_chunk, 0)

    @pl.when(k < 3)
    def _coord():
        run_pass2(k, k + 8, coord=True)

    @pl.when(k >= 3)
    def _plain():
        run_pass2(k, k + 8, coord=False)

    run_pass2(k + 16, k + 24, coord=False)

    @pl.when(k < 3)
    def _tail():
        run_pass1(k + 32)


def kernel(xyz, xyz_batch_cnt, new_xyz, new_xyz_batch_cnt, features):
    xyzt = jnp.transpose(xyz).reshape(-1)
    nxyzt = jnp.transpose(new_xyz).reshape(-1)
    idx, idx2, cnt = _ball_query(xyzt, nxyzt)
    table = jnp.concatenate(
        [jnp.transpose(xyz), jnp.transpose(features)], axis=0).reshape(-1)
    cent = nxyzt
    grouped = _group(table, cent, idx2, cnt)
    new_features = jnp.transpose(grouped, (1, 0, 2))
    return new_features, idx
